# baseline (speedup 1.0000x reference)
import functools

import jax
import jax.numpy as jnp
from jax import lax
from jax.experimental import pallas as pl
from jax.experimental.pallas import tpu as pltpu
from jax.experimental.pallas import tpu_sc as plsc

ORI_N = 100000
THINK_N = 100000
EMBED_D = 64

NC = 2
NS = 16
NW = NC * NS

SEQ_BLK = 2


def _concat_tables(ori, think):
    rpw = (ORI_N + THINK_N) // NW
    mesh = plsc.VectorSubcoreMesh(core_axis_name="c", subcore_axis_name="s")

    @functools.partial(
        pl.kernel,
        out_type=jax.ShapeDtypeStruct((ORI_N + THINK_N, EMBED_D), jnp.float32),
        mesh=mesh,
        scratch_types=[pltpu.SemaphoreType.DMA],
        compiler_params=pltpu.CompilerParams(use_tc_tiling_on_sc=False),
    )
    def k(ori_hbm, think_hbm, out_hbm, sem):
        wid = lax.axis_index("s") * NC + lax.axis_index("c")
        r0 = wid * rpw

        @pl.when(r0 < ORI_N)
        def _():
            pltpu.async_copy(ori_hbm.at[pl.ds(r0, rpw)],
                             out_hbm.at[pl.ds(r0, rpw)], sem)
            pltpu.make_async_copy(ori_hbm.at[pl.ds(r0, rpw)],
                                  out_hbm.at[pl.ds(r0, rpw)], sem).wait()

        @pl.when(r0 >= ORI_N)
        def _():
            r1 = r0 - ORI_N
            pltpu.async_copy(think_hbm.at[pl.ds(r1, rpw)],
                             out_hbm.at[pl.ds(r0, rpw)], sem)
            pltpu.make_async_copy(think_hbm.at[pl.ds(r1, rpw)],
                                  out_hbm.at[pl.ds(r0, rpw)], sem).wait()

    return k(ori, think)


def _gather_kernel(n_seq, seq_len):
    NB = SEQ_BLK * seq_len
    spw = n_seq // NW
    C = spw * seq_len
    T = spw // SEQ_BLK
    assert n_seq % NW == 0 and spw % SEQ_BLK == 0
    mesh = plsc.VectorSubcoreMesh(core_axis_name="c", subcore_axis_name="s")

    @functools.partial(
        pl.kernel,
        out_type=jax.ShapeDtypeStruct((n_seq, seq_len, EMBED_D), jnp.float32),
        mesh=mesh,
        scratch_types=[
            pltpu.VMEM((2, NB), jnp.int32),
            pltpu.VMEM((2, NB, EMBED_D), jnp.float32),
            [pltpu.SemaphoreType.DMA] * 2,
            [pltpu.SemaphoreType.DMA] * 2,
            [pltpu.SemaphoreType.DMA] * 2,
        ],
        compiler_params=pltpu.CompilerParams(use_tc_tiling_on_sc=False),
    )
    def k(ids_hbm, table_hbm, out_hbm, idx_v, rows_v, idsem, gsem, outsem):
        wid = lax.axis_index("s") * NC + lax.axis_index("c")
        base = wid * C
        seq0 = wid * spw

        def idload(t, b):
            pltpu.async_copy(ids_hbm.at[pl.ds(base + t * NB, NB)],
                             idx_v.at[b], idsem[b])

        def gather(t, b):
            del t
            pltpu.async_copy(table_hbm.at[idx_v.at[b]], rows_v.at[b], gsem[b])

        def outcopy(t, b):
            for s in range(SEQ_BLK):
                pltpu.async_copy(rows_v.at[b].at[pl.ds(s * seq_len, seq_len)],
                                 out_hbm.at[seq0 + t * SEQ_BLK + s],
                                 outsem[b])

        def wait_id(b):
            pltpu.make_async_copy(ids_hbm.at[pl.ds(0, NB)], idx_v.at[b],
                                  idsem[b]).wait()

        def wait_g(b):
            pltpu.make_async_copy(table_hbm.at[pl.ds(0, NB)], rows_v.at[b],
                                  gsem[b]).wait()

        def wait_out(b):
            for s in range(SEQ_BLK):
                pltpu.make_async_copy(rows_v.at[b].at[pl.ds(0, seq_len)],
                                      out_hbm.at[seq0], outsem[b]).wait()

        idload(0, 0)
        idload(1, 1)
        wait_id(0)
        gather(0, 0)

        def body(i, carry):
            for s in range(2):
                t = 2 * i + 1 + s
                b = (1 + s) % 2
                ob = 1 - b

                @pl.when(t < T)
                def _():
                    @pl.when(t >= 2)
                    def _():
                        wait_out(b)

                    wait_id(b)
                    gather(t, b)
                    wait_g(ob)

                    @pl.when(t + 1 < T)
                    def _():
                        idload(t + 1, ob)

                    outcopy(t - 1, ob)

            return carry

        lax.fori_loop(0, T // 2, body, 0)

        lb = (T - 1) % 2
        wait_g(lb)
        outcopy(T - 1, lb)
        wait_out(1 - lb)
        wait_out(lb)

    return k


def kernel(ids, ori_weight, think_weight):
    table = _concat_tables(ori_weight, think_weight)
    ids_flat = ids.reshape(-1).astype(jnp.int32)
    return _gather_kernel(ids.shape[0], ids.shape[1])(ids_flat, table)


# speedup vs baseline: 3.8662x; 3.8662x over previous
import functools

import jax
import jax.numpy as jnp
from jax import lax
from jax.experimental import pallas as pl
from jax.experimental.pallas import tpu as pltpu
from jax.experimental.pallas import tpu_sc as plsc

ORI_N = 100000
THINK_N = 100000
EMBED_D = 64
PAD_D = 128

NC = 2
NS = 16
NW = NC * NS

SUB = 128
IDR = 8
GIDS = IDR * 128
SPG = GIDS // SUB


def _gather_kernel(B):
    C = B // NW
    G = C // GIDS
    K = C // SUB
    assert B % NW == 0 and C % GIDS == 0 and K % 2 == 0
    mesh = plsc.VectorSubcoreMesh(core_axis_name="c", subcore_axis_name="s")

    @functools.partial(
        pl.kernel,
        out_type=jax.ShapeDtypeStruct((B, PAD_D), jnp.float32),
        mesh=mesh,
        scratch_types=[
            pltpu.VMEM((2, IDR, 128), jnp.int32),
            pltpu.VMEM((2, SUB, PAD_D), jnp.float32),
            [pltpu.SemaphoreType.DMA] * 2,
            [pltpu.SemaphoreType.DMA] * 2,
            [pltpu.SemaphoreType.DMA] * 2,
        ],
        compiler_params=pltpu.CompilerParams(use_tc_tiling_on_sc=True),
    )
    def k(ids_hbm, table_hbm, out_hbm, idx_v, rows_v, idsem, gsem, outsem):
        wid = lax.axis_index("s") * NC + lax.axis_index("c")
        base = wid * C
        rbase = base // 128

        def idload(g, b):
            off = pl.multiple_of(rbase + g * IDR, 8)
            pltpu.async_copy(ids_hbm.at[pl.ds(off, IDR)],
                             idx_v.at[b], idsem[b])

        def gather(gb, j, rb):
            pltpu.async_copy(table_hbm.at[idx_v.at[gb].at[j]],
                             rows_v.at[rb], gsem[rb])

        def outcopy(k_off, rb):
            off = pl.multiple_of(base + k_off * SUB, SUB)
            pltpu.async_copy(rows_v.at[rb], out_hbm.at[pl.ds(off, SUB)],
                             outsem[rb])

        def wait_id(b):
            pltpu.make_async_copy(ids_hbm.at[pl.ds(0, IDR)], idx_v.at[b],
                                  idsem[b]).wait()

        def wait_g(rb):
            pltpu.make_async_copy(table_hbm.at[pl.ds(0, SUB)], rows_v.at[rb],
                                  gsem[rb]).wait()

        def wait_out(rb):
            pltpu.make_async_copy(rows_v.at[rb],
                                  out_hbm.at[pl.ds(0, SUB)],
                                  outsem[rb]).wait()

        idload(0, 0)

        def body(i, carry):
            for s in range(2):
                g = 2 * i + s
                gb = s

                @pl.when(g < G)
                def _():
                    wait_id(gb)

                    for j in range(SPG):
                        rb = j % 2

                        if j == 1:
                            @pl.when(g + 1 < G)
                            def _():
                                idload(g + 1, 1 - gb)

                        if j >= 2:
                            wait_out(rb)
                        else:
                            @pl.when(g >= 1)
                            def _():
                                wait_out(rb)

                        gather(gb, j, rb)

                        ob = 1 - rb
                        if j >= 1:
                            wait_g(ob)
                            outcopy(g * SPG + j - 1, ob)
                        else:
                            @pl.when(g >= 1)
                            def _():
                                wait_g(ob)
                                outcopy(g * SPG - 1, ob)

            return carry

        lax.fori_loop(0, (G + 1) // 2, body, 0)

        lb = (K - 1) % 2
        wait_g(lb)
        outcopy(K - 1, lb)
        wait_out(1 - lb)
        wait_out(lb)

    return k


def kernel(ids, ori_weight, think_weight):
    table = jnp.concatenate([ori_weight, think_weight], axis=0)
    table = jnp.pad(table, ((0, 0), (0, PAD_D - EMBED_D)))
    ids2 = ids.reshape(-1, 128).astype(jnp.int32)
    out = _gather_kernel(ids.size)(ids2, table)
    return out[:, :EMBED_D].reshape(ids.shape + (EMBED_D,))
